# baseline (speedup 1.0000x reference)
import jax
import jax.numpy as jnp
from jax.experimental import pallas as pl
from jax.experimental.pallas import tpu as pltpu

_CHUNK = 8192
_NBUF = 2


def _linear_dma_kernel(x_hbm, w_ref, b_ref, o_hbm, xbuf, obuf, in_sem, out_sem):
    nchunk = x_hbm.shape[0] // _CHUNK

    def in_copy(c):
        slot = c % _NBUF
        return pltpu.make_async_copy(
            x_hbm.at[pl.ds(c * _CHUNK, _CHUNK), :],
            xbuf.at[slot],
            in_sem.at[slot],
        )

    def out_copy(c):
        slot = c % _NBUF
        return pltpu.make_async_copy(
            obuf.at[slot],
            o_hbm.at[pl.ds(c * _CHUNK, _CHUNK), :],
            out_sem.at[slot],
        )

    for c in range(_NBUF):
        in_copy(c).start()
    for c in range(nchunk):
        slot = c % _NBUF
        in_copy(c).wait()
        if c >= _NBUF:
            out_copy(c - _NBUF).wait()
        obuf[slot] = (
            jax.lax.dot_general(
                xbuf[slot],
                w_ref[...],
                (((1,), (1,)), ((), ())),
                preferred_element_type=jnp.float32,
            )
            + b_ref[...]
        )
        out_copy(c).start()
        if c + _NBUF < nchunk:
            in_copy(c + _NBUF).start()
    for c in range(nchunk - _NBUF, nchunk):
        out_copy(c).wait()


def kernel(speakers, spker_embeds, W, b):
    del speakers
    M, K = spker_embeds.shape
    N = W.shape[0]

    b2 = b.reshape(1, N)

    return pl.pallas_call(
        _linear_dma_kernel,
        in_specs=[
            pl.BlockSpec(memory_space=pltpu.MemorySpace.HBM),
            pl.BlockSpec(memory_space=pltpu.MemorySpace.VMEM),
            pl.BlockSpec(memory_space=pltpu.MemorySpace.VMEM),
        ],
        out_specs=pl.BlockSpec(memory_space=pltpu.MemorySpace.HBM),
        out_shape=jax.ShapeDtypeStruct((M, N), jnp.float32),
        scratch_shapes=[
            pltpu.VMEM((_NBUF, _CHUNK, K), jnp.float32),
            pltpu.VMEM((_NBUF, _CHUNK, N), jnp.float32),
            pltpu.SemaphoreType.DMA((_NBUF,)),
            pltpu.SemaphoreType.DMA((_NBUF,)),
        ],
    )(spker_embeds, W, b2)


# speedup vs baseline: 1.1394x; 1.1394x over previous
import jax
import jax.numpy as jnp
from jax.experimental import pallas as pl
from jax.experimental.pallas import tpu as pltpu


def _linear_kernel(x_ref, w_ref, b_ref, o_ref):
    o_ref[...] = (
        jax.lax.dot_general(
            x_ref[...],
            w_ref[...],
            (((1,), (1,)), ((), ())),
            preferred_element_type=jnp.float32,
        )
        + b_ref[...]
    )


def kernel(speakers, spker_embeds, W, b):
    del speakers
    M, K = spker_embeds.shape
    N = W.shape[0]
    BM = 8192

    b2 = b.reshape(1, N)

    return pl.pallas_call(
        _linear_kernel,
        grid=(M // BM,),
        in_specs=[
            pl.BlockSpec((BM, K), lambda i: (i, 0)),
            pl.BlockSpec((N, K), lambda i: (0, 0)),
            pl.BlockSpec((1, N), lambda i: (0, 0)),
        ],
        out_specs=pl.BlockSpec((BM, N), lambda i: (i, 0)),
        out_shape=jax.ShapeDtypeStruct((M, N), jnp.float32),
        compiler_params=pltpu.CompilerParams(
            dimension_semantics=("parallel",),
        ),
    )(spker_embeds, W, b2)
